# baseline (speedup 1.0000x reference)
import functools

import jax
import jax.numpy as jnp
from jax import lax
from jax.experimental import pallas as pl
from jax.experimental.pallas import tpu as pltpu
from jax.experimental.pallas import tpu_sc as plsc

N = 10000
E = 320000
D_IN = 128
D_E = 16
H = 64
C = 3
G = 64

NC = 2
NS = 16
NW = NC * NS
EPW = E // NW
CHUNK = 80
NCHUNK = EPW // CHUNK
NP = 10240
RPS = NP // NS

ROW_BLK = 1000
N_BLKS = N // ROW_BLK


NB = 5
AHEAD = 2


def _ring(phase_fn, drain_fn):
    for j in range(NB):
        phase_fn(j, j, j >= NB - AHEAD, True)

    def lap(k, _):
        j0 = NB * k
        for b in range(NB):
            phase_fn(j0 + b, b, True, True)
        return 0

    lax.fori_loop(1, NCHUNK // NB - 1, lap, 0)
    j0 = NCHUNK - NB
    for b in range(NB):
        phase_fn(j0 + b, b, True, j0 + b + AHEAD < NCHUNK)
    for b in range(AHEAD, NB):
        drain_fn(b)


def _msg_body(with_attr, *refs):
    if with_attr:
        (xw_hbm, src_hbm, dst_hbm, ea_hbm, z64_hbm, z16_hbm, z16f_hbm,
         ones_hbm, out_m, out_e, out_d,
         src_v, dst_v, acc_m, gb, sem_g, sem_s,
         ab, ones_v, acc_e, acc_d, sem_al, sem_as, sem_o) = refs
    else:
        (xw_hbm, src_hbm, dst_hbm, z64_hbm, out_m,
         src_v, dst_v, acc_m, gb, sem_g, sem_s) = refs

    c = lax.axis_index("c")
    s = lax.axis_index("s")
    w = c * NS + s
    r0 = s * RPS
    ebase = w * EPW

    pltpu.sync_copy(z64_hbm.at[pl.ds(r0, RPS)], acc_m.at[pl.ds(r0, RPS)])
    if with_attr:
        pltpu.sync_copy(z16_hbm.at[pl.ds(r0, RPS)], acc_e.at[pl.ds(r0, RPS)])
        pltpu.sync_copy(z16f_hbm.at[pl.ds(r0, RPS)], acc_d.at[pl.ds(r0, RPS)])
        pltpu.sync_copy(ones_hbm, ones_v)
    pltpu.sync_copy(src_hbm.at[w], src_v)
    pltpu.sync_copy(dst_hbm.at[w], dst_v)
    plsc.subcore_barrier()

    def g_fire(j, b):
        pltpu.async_copy(xw_hbm.at[src_v.at[j]], gb.at[b], sem_g[b])

    def g_wait(b):
        pltpu.make_async_copy(xw_hbm.at[src_v.at[0]], gb.at[b], sem_g[b]).wait()

    def s_fire(j, b):
        pltpu.async_copy(gb.at[b], acc_m.at[dst_v.at[j]], sem_s[b], add=True)

    def s_wait(b):
        pltpu.make_async_copy(gb.at[b], acc_m.at[dst_v.at[0]], sem_s[b]).wait()

    if with_attr:
        def al_fire(j, b):
            pltpu.async_copy(ea_hbm.at[pl.ds(ebase + j * CHUNK, CHUNK)],
                             ab.at[b], sem_al[b])

        def al_wait(b):
            pltpu.make_async_copy(ea_hbm.at[pl.ds(ebase, CHUNK)],
                                  ab.at[b], sem_al[b]).wait()

        def as_fire(j, b):
            pltpu.async_copy(ab.at[b], acc_e.at[dst_v.at[j]], sem_as[b],
                             add=True)

        def as_wait(b):
            pltpu.make_async_copy(ab.at[b], acc_e.at[dst_v.at[0]],
                                  sem_as[b]).wait()

        def o_fire(j, b):
            pltpu.async_copy(ones_v, acc_d.at[dst_v.at[j]], sem_o[b], add=True)

        def o_wait(b):
            pltpu.make_async_copy(ones_v, acc_d.at[dst_v.at[0]],
                                  sem_o[b]).wait()

    def phase(j, b, drain, fire):
        b2 = (b + AHEAD) % NB
        g_wait(b)
        if with_attr:
            al_wait(b)
        if drain:
            s_wait(b2)
            if with_attr:
                as_wait(b2)
                o_wait(b2)
        if fire:
            g_fire(j + AHEAD, b2)
            if with_attr:
                al_fire(j + AHEAD, b2)
        s_fire(j, b)
        if with_attr:
            as_fire(j, b)
            o_fire(j, b)

    def drain(b):
        s_wait(b)
        if with_attr:
            as_wait(b)
            o_wait(b)

    for j in range(AHEAD):
        g_fire(j, j)
        if with_attr:
            al_fire(j, j)
    _ring(phase, drain)

    plsc.subcore_barrier()
    pltpu.sync_copy(acc_m.at[pl.ds(r0, RPS)], out_m.at[c, pl.ds(r0, RPS)])
    if with_attr:
        pltpu.sync_copy(acc_e.at[pl.ds(r0, RPS)], out_e.at[c, pl.ds(r0, RPS)])
        pltpu.sync_copy(acc_d.at[pl.ds(r0, RPS)], out_d.at[c, pl.ds(r0, RPS)])


def _attr_body(dst_hbm, ea_hbm, z16_hbm, z16f_hbm, ones_hbm, out_e, out_d,
               dst_v, ab, ones_v, acc_e, acc_d, sem_al, sem_as, sem_o):
    c = lax.axis_index("c")
    s = lax.axis_index("s")
    w = c * NS + s
    r0 = s * RPS
    ebase = w * EPW

    pltpu.sync_copy(z16_hbm.at[pl.ds(r0, RPS)], acc_e.at[pl.ds(r0, RPS)])
    pltpu.sync_copy(z16f_hbm.at[pl.ds(r0, RPS)], acc_d.at[pl.ds(r0, RPS)])
    pltpu.sync_copy(ones_hbm, ones_v)
    pltpu.sync_copy(dst_hbm.at[w], dst_v)
    plsc.subcore_barrier()

    def al_fire(j, b):
        pltpu.async_copy(ea_hbm.at[pl.ds(ebase + j * CHUNK, CHUNK)],
                         ab.at[b], sem_al[b])

    def al_wait(b):
        pltpu.make_async_copy(ea_hbm.at[pl.ds(ebase, CHUNK)],
                              ab.at[b], sem_al[b]).wait()

    def as_fire(j, b):
        pltpu.async_copy(ab.at[b], acc_e.at[dst_v.at[j]], sem_as[b], add=True)

    def as_wait(b):
        pltpu.make_async_copy(ab.at[b], acc_e.at[dst_v.at[0]], sem_as[b]).wait()

    def o_fire(j, b):
        pltpu.async_copy(ones_v, acc_d.at[dst_v.at[j]], sem_o[b], add=True)

    def o_wait(b):
        pltpu.make_async_copy(ones_v, acc_d.at[dst_v.at[0]], sem_o[b]).wait()

    def phase(j, b, drain, fire):
        b2 = (b + AHEAD) % NB
        al_wait(b)
        if drain:
            as_wait(b2)
            o_wait(b2)
        if fire:
            al_fire(j + AHEAD, b2)
        as_fire(j, b)
        o_fire(j, b)

    def drain(b):
        as_wait(b)
        o_wait(b)

    for j in range(AHEAD):
        al_fire(j, j)
    _ring(phase, drain)

    plsc.subcore_barrier()
    pltpu.sync_copy(acc_e.at[pl.ds(r0, RPS)], out_e.at[c, pl.ds(r0, RPS)])
    pltpu.sync_copy(acc_d.at[pl.ds(r0, RPS)], out_d.at[c, pl.ds(r0, RPS)])


def _make_attr_pass():
    mesh = plsc.VectorSubcoreMesh(core_axis_name="c", subcore_axis_name="s")
    f32 = jnp.float32
    bf16 = jnp.bfloat16
    return pl.kernel(
        _attr_body,
        out_type=(jax.ShapeDtypeStruct((NC, NP, D_E), bf16),
                  jax.ShapeDtypeStruct((NC, NP, 16), f32)),
        mesh=mesh,
        scratch_types=(
            pltpu.VMEM((NCHUNK, CHUNK), jnp.int32),
            pltpu.VMEM((NB, CHUNK, D_E), bf16),
            pltpu.VMEM((CHUNK, 16), f32),
            pltpu.VMEM_SHARED((NP, D_E), bf16),
            pltpu.VMEM_SHARED((NP, 16), f32),
            [pltpu.SemaphoreType.DMA] * NB,
            [pltpu.SemaphoreType.DMA] * NB,
            [pltpu.SemaphoreType.DMA] * NB,
        ),
        compiler_params=pltpu.CompilerParams(use_tc_tiling_on_sc=False),
    )


def _make_msg_pass(with_attr):
    mesh = plsc.VectorSubcoreMesh(core_axis_name="c", subcore_axis_name="s")
    f32 = jnp.float32
    bf16 = jnp.bfloat16
    out_type = [jax.ShapeDtypeStruct((NC, NP, H), f32)]
    scratch = [
        pltpu.VMEM((NCHUNK, CHUNK), jnp.int32),
        pltpu.VMEM((NCHUNK, CHUNK), jnp.int32),
        pltpu.VMEM_SHARED((NP, H), f32),
        pltpu.VMEM((NB, CHUNK, H), f32),
        [pltpu.SemaphoreType.DMA] * NB,
        [pltpu.SemaphoreType.DMA] * NB,
    ]
    if with_attr:
        out_type += [jax.ShapeDtypeStruct((NC, NP, D_E), bf16),
                     jax.ShapeDtypeStruct((NC, NP, 16), f32)]
        scratch += [
            pltpu.VMEM((NB, CHUNK, D_E), bf16),
            pltpu.VMEM((CHUNK, 16), f32),
            pltpu.VMEM_SHARED((NP, D_E), bf16),
            pltpu.VMEM_SHARED((NP, 16), f32),
            [pltpu.SemaphoreType.DMA] * NB,
            [pltpu.SemaphoreType.DMA] * NB,
            [pltpu.SemaphoreType.DMA] * NB,
        ]
    return pl.kernel(
        functools.partial(_msg_body, with_attr),
        out_type=tuple(out_type),
        mesh=mesh,
        scratch_types=tuple(scratch),
        compiler_params=pltpu.CompilerParams(use_tc_tiling_on_sc=False),
    )


def _tc0_body(x_ref, wx_ref, ws_ref, bs_ref, xw_out, selfx_out):
    x = x_ref[...]
    xw_out[...] = jnp.dot(x, wx_ref[...], preferred_element_type=jnp.float32)
    selfx_out[...] = (
        jnp.dot(x, ws_ref[...], preferred_element_type=jnp.float32) + bs_ref[...]
    )


def _tc_mid_body(selfx_ref, accm_ref, acce_ref, accd_ref,
                 we_ref, bm_ref, wh2_ref, ws2_ref, bs2_ref,
                 hw2_out, self2_out):
    am = accm_ref[0] + accm_ref[1]
    ae = (acce_ref[0] + acce_ref[1]).astype(jnp.float32)
    deg = accd_ref[0, :, 0:1] + accd_ref[1, :, 0:1]
    dc = jnp.maximum(deg, 1.0)
    ind = jnp.minimum(deg, 1.0)
    agg = (am + jnp.dot(ae, we_ref[...], preferred_element_type=jnp.float32)) / dc
    h1 = jnp.maximum(selfx_ref[...] + agg + ind * bm_ref[...], 0.0)
    hw2_out[...] = jnp.dot(h1, wh2_ref[...], preferred_element_type=jnp.float32)
    self2_out[...] = (
        jnp.dot(h1, ws2_ref[...], preferred_element_type=jnp.float32) + bs2_ref[...]
    )


def _tc_final_body(self2_ref, accm_ref, acce_ref, accd_ref, batch_ref,
                   we_ref, bm_ref, wfc_ref, bfc_ref,
                   out_ref, pooled_acc, cnt_acc):
    i = pl.program_id(0)

    am = accm_ref[0] + accm_ref[1]
    ae = (acce_ref[0] + acce_ref[1]).astype(jnp.float32)
    deg = accd_ref[0, :, 0:1] + accd_ref[1, :, 0:1]
    dc = jnp.maximum(deg, 1.0)
    ind = jnp.minimum(deg, 1.0)
    agg = (am + jnp.dot(ae, we_ref[...], preferred_element_type=jnp.float32)) / dc
    h2 = jnp.maximum(self2_ref[...] + agg + ind * bm_ref[...], 0.0)

    gids = lax.broadcasted_iota(jnp.int32, (ROW_BLK, G), 1)
    onehot = (batch_ref[...] == gids).astype(jnp.float32)
    pooled = lax.dot_general(onehot, h2, (((0,), (0,)), ((), ())),
                             preferred_element_type=jnp.float32)
    cnt = lax.dot_general(onehot, jnp.ones((ROW_BLK, 1), jnp.float32),
                          (((0,), (0,)), ((), ())),
                          preferred_element_type=jnp.float32)

    @pl.when(i == 0)
    def _():
        pooled_acc[...] = jnp.zeros_like(pooled_acc)
        cnt_acc[...] = jnp.zeros_like(cnt_acc)

    pooled_acc[...] += pooled
    cnt_acc[...] += cnt

    @pl.when(i == N_BLKS - 1)
    def _():
        mean = pooled_acc[...] / jnp.maximum(cnt_acc[...], 1.0)
        out_ref[...] = (
            jnp.dot(mean, wfc_ref[...], preferred_element_type=jnp.float32)
            + bfc_ref[...]
        )


def _row_spec(width):
    return pl.BlockSpec((ROW_BLK, width), lambda i: (i, 0))


def _part_spec(width):
    return pl.BlockSpec((NC, ROW_BLK, width), lambda i: (0, i, 0))


def _full_spec(shape):
    return pl.BlockSpec(shape, lambda i: tuple(0 for _ in shape))


@jax.jit
def kernel(x, edge_index, edge_attr, batch, W_msg1, b_msg1, W_self1, b_self1,
           W_msg2, b_msg2, W_self2, b_self2, W_fc, b_fc):
    f32 = jnp.float32
    src = edge_index[0].reshape(NW, NCHUNK, CHUNK)
    dst = edge_index[1].reshape(NW, NCHUNK, CHUNK)
    Wx1 = W_msg1[:D_IN]
    We1 = W_msg1[D_IN:]
    Wh2 = W_msg2[:H]
    We2 = W_msg2[H:]
    z64 = jnp.zeros((NP, H), f32)
    z16bf = jnp.zeros((NP, 16), jnp.bfloat16)
    z16f = jnp.zeros((NP, 16), f32)
    ones_col = jnp.zeros((CHUNK, 16), f32).at[:, 0].set(1.0)
    batch2d = batch.reshape(N, 1)

    xw1, selfx = pl.pallas_call(
        _tc0_body,
        grid=(N_BLKS,),
        in_specs=[_row_spec(D_IN), _full_spec((D_IN, H)), _full_spec((D_IN, H)),
                  _full_spec((1, H))],
        out_specs=[_row_spec(H), _row_spec(H)],
        out_shape=[jax.ShapeDtypeStruct((N, H), f32),
                   jax.ShapeDtypeStruct((N, H), f32)],
    )(x, Wx1, W_self1, b_self1.reshape(1, H))

    ea_bf = edge_attr.astype(jnp.bfloat16)
    (accm1,) = _make_msg_pass(False)(xw1, src, dst, z64)
    acce, accd = _make_attr_pass()(dst, ea_bf, z16bf, z16f, ones_col)

    hw2, self2 = pl.pallas_call(
        _tc_mid_body,
        grid=(N_BLKS,),
        in_specs=[_row_spec(H), _part_spec(H), _part_spec(D_E), _part_spec(16),
                  _full_spec((D_E, H)), _full_spec((1, H)),
                  _full_spec((H, H)), _full_spec((H, H)), _full_spec((1, H))],
        out_specs=[_row_spec(H), _row_spec(H)],
        out_shape=[jax.ShapeDtypeStruct((N, H), f32),
                   jax.ShapeDtypeStruct((N, H), f32)],
    )(selfx, accm1, acce, accd, We1, b_msg1.reshape(1, H),
      Wh2, W_self2, b_self2.reshape(1, H))

    (accm2,) = _make_msg_pass(False)(hw2, src, dst, z64)

    out = pl.pallas_call(
        _tc_final_body,
        grid=(N_BLKS,),
        in_specs=[_row_spec(H), _part_spec(H), _part_spec(D_E), _part_spec(16),
                  pl.BlockSpec((ROW_BLK, 1), lambda i: (i, 0)),
                  _full_spec((D_E, H)), _full_spec((1, H)),
                  _full_spec((H, C)), _full_spec((1, C))],
        out_specs=pl.BlockSpec((G, C), lambda i: (0, 0)),
        out_shape=jax.ShapeDtypeStruct((G, C), f32),
        scratch_shapes=[pltpu.VMEM((G, H), f32), pltpu.VMEM((G, 1), f32)],
    )(self2, accm2, acce, accd, batch2d, We2, b_msg2.reshape(1, H),
      W_fc, b_fc.reshape(1, C))

    return out


# speedup vs baseline: 1.2011x; 1.2011x over previous
import functools

import jax
import jax.numpy as jnp
from jax import lax
from jax.experimental import pallas as pl
from jax.experimental.pallas import tpu as pltpu
from jax.experimental.pallas import tpu_sc as plsc

N = 10000
E = 320000
D_IN = 128
D_E = 16
H = 64
C = 3
G = 64

NC = 2
NS = 16
NW = NC * NS
EPW = E // NW
CHUNK = 80
NCHUNK = EPW // CHUNK
NP = 10240
RPS = NP // NS

ROW_BLK = 1000
N_BLKS = N // ROW_BLK


NB = 5
AHEAD = 2


def _ring(phase_fn, drain_fn):
    for j in range(NB):
        phase_fn(j, j, j >= NB - AHEAD, True)

    def lap(k, _):
        j0 = NB * k
        for b in range(NB):
            phase_fn(j0 + b, b, True, True)
        return 0

    lax.fori_loop(1, NCHUNK // NB - 1, lap, 0)
    j0 = NCHUNK - NB
    for b in range(NB):
        phase_fn(j0 + b, b, True, j0 + b + AHEAD < NCHUNK)
    for b in range(AHEAD, NB):
        drain_fn(b)


def _msg_body(with_attr, *refs):
    if with_attr:
        (xw_hbm, src_hbm, dst_hbm, ea_hbm, z64_hbm, z16_hbm, z16f_hbm,
         ones_hbm, out_m, out_e, out_d,
         src_v, dst_v, acc_m, gb, sem_g, sem_s,
         ab, ones_v, acc_e, acc_d, sem_al, sem_as, sem_o) = refs
    else:
        (xw_hbm, src_hbm, dst_hbm, z64_hbm, out_m,
         src_v, dst_v, acc_m, gb, sem_g, sem_s) = refs

    c = lax.axis_index("c")
    s = lax.axis_index("s")
    w = c * NS + s
    r0 = s * RPS
    ebase = w * EPW

    pltpu.sync_copy(z64_hbm.at[pl.ds(r0, RPS)], acc_m.at[pl.ds(r0, RPS)])
    if with_attr:
        pltpu.sync_copy(z16_hbm.at[pl.ds(r0, RPS)], acc_e.at[pl.ds(r0, RPS)])
        pltpu.sync_copy(z16f_hbm.at[pl.ds(r0, RPS)], acc_d.at[pl.ds(r0, RPS)])
        pltpu.sync_copy(ones_hbm, ones_v)
    pltpu.sync_copy(src_hbm.at[w], src_v)
    pltpu.sync_copy(dst_hbm.at[w], dst_v)
    plsc.subcore_barrier()

    def g_fire(j, b):
        pltpu.async_copy(xw_hbm.at[src_v.at[j]], gb.at[b], sem_g[b])

    def g_wait(b):
        pltpu.make_async_copy(xw_hbm.at[src_v.at[0]], gb.at[b], sem_g[b]).wait()

    def s_fire(j, b):
        pltpu.async_copy(gb.at[b], acc_m.at[dst_v.at[j]], sem_s[b], add=True)

    def s_wait(b):
        pltpu.make_async_copy(gb.at[b], acc_m.at[dst_v.at[0]], sem_s[b]).wait()

    if with_attr:
        def al_fire(j, b):
            pltpu.async_copy(ea_hbm.at[pl.ds(ebase + j * CHUNK, CHUNK)],
                             ab.at[b], sem_al[b])

        def al_wait(b):
            pltpu.make_async_copy(ea_hbm.at[pl.ds(ebase, CHUNK)],
                                  ab.at[b], sem_al[b]).wait()

        def as_fire(j, b):
            pltpu.async_copy(ab.at[b], acc_e.at[dst_v.at[j]], sem_as[b],
                             add=True)

        def as_wait(b):
            pltpu.make_async_copy(ab.at[b], acc_e.at[dst_v.at[0]],
                                  sem_as[b]).wait()

        def o_fire(j, b):
            pltpu.async_copy(ones_v, acc_d.at[dst_v.at[j]], sem_o[b], add=True)

        def o_wait(b):
            pltpu.make_async_copy(ones_v, acc_d.at[dst_v.at[0]],
                                  sem_o[b]).wait()

    def phase(j, b, drain, fire):
        b2 = (b + AHEAD) % NB
        g_wait(b)
        if with_attr:
            al_wait(b)
        if drain:
            s_wait(b2)
            if with_attr:
                as_wait(b2)
                o_wait(b2)
        if fire:
            g_fire(j + AHEAD, b2)
            if with_attr:
                al_fire(j + AHEAD, b2)
        s_fire(j, b)
        if with_attr:
            as_fire(j, b)
            o_fire(j, b)

    def drain(b):
        s_wait(b)
        if with_attr:
            as_wait(b)
            o_wait(b)

    for j in range(AHEAD):
        g_fire(j, j)
        if with_attr:
            al_fire(j, j)
    _ring(phase, drain)

    plsc.subcore_barrier()
    pltpu.sync_copy(acc_m.at[pl.ds(r0, RPS)], out_m.at[c, pl.ds(r0, RPS)])
    if with_attr:
        pltpu.sync_copy(acc_e.at[pl.ds(r0, RPS)], out_e.at[c, pl.ds(r0, RPS)])
        pltpu.sync_copy(acc_d.at[pl.ds(r0, RPS)], out_d.at[c, pl.ds(r0, RPS)])


def _attr_body(dst_hbm, ea_hbm, z16f_hbm, ones_hbm, out_e, out_d,
               dst_v, ab, ones_v, acc_e, acc_d, sem_al, sem_as, sem_o):
    c = lax.axis_index("c")
    s = lax.axis_index("s")
    w = c * NS + s
    r0 = s * RPS
    ebase = w * EPW

    pltpu.sync_copy(z16f_hbm.at[pl.ds(r0, RPS)], acc_e.at[pl.ds(r0, RPS)])
    pltpu.sync_copy(z16f_hbm.at[pl.ds(r0, RPS)], acc_d.at[pl.ds(r0, RPS)])
    pltpu.sync_copy(ones_hbm, ones_v)
    pltpu.sync_copy(dst_hbm.at[w], dst_v)
    plsc.subcore_barrier()

    def al_fire(j, b):
        pltpu.async_copy(ea_hbm.at[pl.ds(ebase + j * CHUNK, CHUNK)],
                         ab.at[b], sem_al[b])

    def al_wait(b):
        pltpu.make_async_copy(ea_hbm.at[pl.ds(ebase, CHUNK)],
                              ab.at[b], sem_al[b]).wait()

    def as_fire(j, b):
        pltpu.async_copy(ab.at[b], acc_e.at[dst_v.at[j]], sem_as[b], add=True)

    def as_wait(b):
        pltpu.make_async_copy(ab.at[b], acc_e.at[dst_v.at[0]], sem_as[b]).wait()

    def o_fire(j, b):
        pltpu.async_copy(ones_v, acc_d.at[dst_v.at[j]], sem_o[b], add=True)

    def o_wait(b):
        pltpu.make_async_copy(ones_v, acc_d.at[dst_v.at[0]], sem_o[b]).wait()

    def phase(j, b, drain, fire):
        b2 = (b + AHEAD) % NB
        al_wait(b)
        if drain:
            as_wait(b2)
            o_wait(b2)
        if fire:
            al_fire(j + AHEAD, b2)
        as_fire(j, b)
        o_fire(j, b)

    def drain(b):
        as_wait(b)
        o_wait(b)

    for j in range(AHEAD):
        al_fire(j, j)
    _ring(phase, drain)

    plsc.subcore_barrier()
    pltpu.sync_copy(acc_e.at[pl.ds(r0, RPS)], out_e.at[c, pl.ds(r0, RPS)])
    pltpu.sync_copy(acc_d.at[pl.ds(r0, RPS)], out_d.at[c, pl.ds(r0, RPS)])


def _make_attr_pass():
    mesh = plsc.VectorSubcoreMesh(core_axis_name="c", subcore_axis_name="s")
    f32 = jnp.float32
    bf16 = jnp.bfloat16
    return pl.kernel(
        _attr_body,
        out_type=(jax.ShapeDtypeStruct((NC, NP, D_E), f32),
                  jax.ShapeDtypeStruct((NC, NP, 16), f32)),
        mesh=mesh,
        scratch_types=(
            pltpu.VMEM((NCHUNK, CHUNK), jnp.int32),
            pltpu.VMEM((NB, CHUNK, D_E), f32),
            pltpu.VMEM((CHUNK, 16), f32),
            pltpu.VMEM_SHARED((NP, D_E), f32),
            pltpu.VMEM_SHARED((NP, 16), f32),
            [pltpu.SemaphoreType.DMA] * NB,
            [pltpu.SemaphoreType.DMA] * NB,
            [pltpu.SemaphoreType.DMA] * NB,
        ),
        compiler_params=pltpu.CompilerParams(use_tc_tiling_on_sc=False),
    )


def _make_msg_pass(with_attr):
    mesh = plsc.VectorSubcoreMesh(core_axis_name="c", subcore_axis_name="s")
    f32 = jnp.float32
    bf16 = jnp.bfloat16
    out_type = [jax.ShapeDtypeStruct((NC, NP, H), f32)]
    scratch = [
        pltpu.VMEM((NCHUNK, CHUNK), jnp.int32),
        pltpu.VMEM((NCHUNK, CHUNK), jnp.int32),
        pltpu.VMEM_SHARED((NP, H), f32),
        pltpu.VMEM((NB, CHUNK, H), f32),
        [pltpu.SemaphoreType.DMA] * NB,
        [pltpu.SemaphoreType.DMA] * NB,
    ]
    if with_attr:
        out_type += [jax.ShapeDtypeStruct((NC, NP, D_E), bf16),
                     jax.ShapeDtypeStruct((NC, NP, 16), f32)]
        scratch += [
            pltpu.VMEM((NB, CHUNK, D_E), bf16),
            pltpu.VMEM((CHUNK, 16), f32),
            pltpu.VMEM_SHARED((NP, D_E), bf16),
            pltpu.VMEM_SHARED((NP, 16), f32),
            [pltpu.SemaphoreType.DMA] * NB,
            [pltpu.SemaphoreType.DMA] * NB,
            [pltpu.SemaphoreType.DMA] * NB,
        ]
    return pl.kernel(
        functools.partial(_msg_body, with_attr),
        out_type=tuple(out_type),
        mesh=mesh,
        scratch_types=tuple(scratch),
        compiler_params=pltpu.CompilerParams(use_tc_tiling_on_sc=False),
    )


def _tc0_body(x_ref, wx_ref, ws_ref, bs_ref, xw_out, selfx_out):
    x = x_ref[...]
    xw_out[...] = jnp.dot(x, wx_ref[...], preferred_element_type=jnp.float32)
    selfx_out[...] = (
        jnp.dot(x, ws_ref[...], preferred_element_type=jnp.float32) + bs_ref[...]
    )


def _tc_mid_body(selfx_ref, accm_ref, acce_ref, accd_ref,
                 we_ref, bm_ref, wh2_ref, ws2_ref, bs2_ref,
                 hw2_out, self2_out):
    am = accm_ref[0] + accm_ref[1]
    ae = (acce_ref[0] + acce_ref[1]).astype(jnp.float32)
    deg = accd_ref[0, :, 0:1] + accd_ref[1, :, 0:1]
    dc = jnp.maximum(deg, 1.0)
    ind = jnp.minimum(deg, 1.0)
    agg = (am + jnp.dot(ae, we_ref[...], preferred_element_type=jnp.float32)) / dc
    h1 = jnp.maximum(selfx_ref[...] + agg + ind * bm_ref[...], 0.0)
    hw2_out[...] = jnp.dot(h1, wh2_ref[...], preferred_element_type=jnp.float32)
    self2_out[...] = (
        jnp.dot(h1, ws2_ref[...], preferred_element_type=jnp.float32) + bs2_ref[...]
    )


def _tc_final_body(self2_ref, accm_ref, acce_ref, accd_ref, batch_ref,
                   we_ref, bm_ref, wfc_ref, bfc_ref,
                   out_ref, pooled_acc, cnt_acc):
    i = pl.program_id(0)

    am = accm_ref[0] + accm_ref[1]
    ae = (acce_ref[0] + acce_ref[1]).astype(jnp.float32)
    deg = accd_ref[0, :, 0:1] + accd_ref[1, :, 0:1]
    dc = jnp.maximum(deg, 1.0)
    ind = jnp.minimum(deg, 1.0)
    agg = (am + jnp.dot(ae, we_ref[...], preferred_element_type=jnp.float32)) / dc
    h2 = jnp.maximum(self2_ref[...] + agg + ind * bm_ref[...], 0.0)

    gids = lax.broadcasted_iota(jnp.int32, (ROW_BLK, G), 1)
    onehot = (batch_ref[...] == gids).astype(jnp.float32)
    pooled = lax.dot_general(onehot, h2, (((0,), (0,)), ((), ())),
                             preferred_element_type=jnp.float32)
    cnt = lax.dot_general(onehot, jnp.ones((ROW_BLK, 1), jnp.float32),
                          (((0,), (0,)), ((), ())),
                          preferred_element_type=jnp.float32)

    @pl.when(i == 0)
    def _():
        pooled_acc[...] = jnp.zeros_like(pooled_acc)
        cnt_acc[...] = jnp.zeros_like(cnt_acc)

    pooled_acc[...] += pooled
    cnt_acc[...] += cnt

    @pl.when(i == N_BLKS - 1)
    def _():
        mean = pooled_acc[...] / jnp.maximum(cnt_acc[...], 1.0)
        out_ref[...] = (
            jnp.dot(mean, wfc_ref[...], preferred_element_type=jnp.float32)
            + bfc_ref[...]
        )


def _row_spec(width):
    return pl.BlockSpec((ROW_BLK, width), lambda i: (i, 0))


def _part_spec(width):
    return pl.BlockSpec((NC, ROW_BLK, width), lambda i: (0, i, 0))


def _full_spec(shape):
    return pl.BlockSpec(shape, lambda i: tuple(0 for _ in shape))


@jax.jit
def kernel(x, edge_index, edge_attr, batch, W_msg1, b_msg1, W_self1, b_self1,
           W_msg2, b_msg2, W_self2, b_self2, W_fc, b_fc):
    f32 = jnp.float32
    src = edge_index[0].reshape(NW, NCHUNK, CHUNK)
    dst = edge_index[1].reshape(NW, NCHUNK, CHUNK)
    Wx1 = W_msg1[:D_IN]
    We1 = W_msg1[D_IN:]
    Wh2 = W_msg2[:H]
    We2 = W_msg2[H:]
    z64 = jnp.zeros((NP, H), f32)
    z16f = jnp.zeros((NP, 16), f32)
    ones_col = jnp.zeros((CHUNK, 16), f32).at[:, 0].set(1.0)
    batch2d = batch.reshape(N, 1)

    xw1, selfx = pl.pallas_call(
        _tc0_body,
        grid=(N_BLKS,),
        in_specs=[_row_spec(D_IN), _full_spec((D_IN, H)), _full_spec((D_IN, H)),
                  _full_spec((1, H))],
        out_specs=[_row_spec(H), _row_spec(H)],
        out_shape=[jax.ShapeDtypeStruct((N, H), f32),
                   jax.ShapeDtypeStruct((N, H), f32)],
    )(x, Wx1, W_self1, b_self1.reshape(1, H))

    (accm1,) = _make_msg_pass(False)(xw1, src, dst, z64)
    acce, accd = _make_attr_pass()(dst, edge_attr, z16f, ones_col)

    hw2, self2 = pl.pallas_call(
        _tc_mid_body,
        grid=(N_BLKS,),
        in_specs=[_row_spec(H), _part_spec(H), _part_spec(D_E), _part_spec(16),
                  _full_spec((D_E, H)), _full_spec((1, H)),
                  _full_spec((H, H)), _full_spec((H, H)), _full_spec((1, H))],
        out_specs=[_row_spec(H), _row_spec(H)],
        out_shape=[jax.ShapeDtypeStruct((N, H), f32),
                   jax.ShapeDtypeStruct((N, H), f32)],
    )(selfx, accm1, acce, accd, We1, b_msg1.reshape(1, H),
      Wh2, W_self2, b_self2.reshape(1, H))

    (accm2,) = _make_msg_pass(False)(hw2, src, dst, z64)

    out = pl.pallas_call(
        _tc_final_body,
        grid=(N_BLKS,),
        in_specs=[_row_spec(H), _part_spec(H), _part_spec(D_E), _part_spec(16),
                  pl.BlockSpec((ROW_BLK, 1), lambda i: (i, 0)),
                  _full_spec((D_E, H)), _full_spec((1, H)),
                  _full_spec((H, C)), _full_spec((1, C))],
        out_specs=pl.BlockSpec((G, C), lambda i: (0, 0)),
        out_shape=jax.ShapeDtypeStruct((G, C), f32),
        scratch_shapes=[pltpu.VMEM((G, H), f32), pltpu.VMEM((G, 1), f32)],
    )(self2, accm2, acce, accd, batch2d, We2, b_msg2.reshape(1, H),
      W_fc, b_fc.reshape(1, C))

    return out


# speedup vs baseline: 1.2800x; 1.0657x over previous
import functools

import jax
import jax.numpy as jnp
from jax import lax
from jax.experimental import pallas as pl
from jax.experimental.pallas import tpu as pltpu
from jax.experimental.pallas import tpu_sc as plsc

N = 10000
E = 320000
D_IN = 128
D_E = 16
H = 64
C = 3
G = 64

NC = 2
NS = 16
NW = NC * NS
EPW = E // NW
CHUNK = 80
NCHUNK = EPW // CHUNK
NP = 10240
RPS = NP // NS

ROW_BLK = 1000
N_BLKS = N // ROW_BLK


NB = 5
AHEAD = 2


def _ring(phase_fn, drain_fn):
    for j in range(NB):
        phase_fn(j, j, j >= NB - AHEAD, True)

    def lap(k, _):
        j0 = NB * k
        for b in range(NB):
            phase_fn(j0 + b, b, True, True)
        return 0

    lax.fori_loop(1, NCHUNK // NB - 1, lap, 0)
    j0 = NCHUNK - NB
    for b in range(NB):
        phase_fn(j0 + b, b, True, j0 + b + AHEAD < NCHUNK)
    for b in range(AHEAD, NB):
        drain_fn(b)


def _msg_body(xw_hbm, src_hbm, dst_hbm, z64_hbm, out_m,
              src_v, dst_v, acc_m, gb, sem_g, sem_s):
    c = lax.axis_index("c")
    s = lax.axis_index("s")
    w = c * NS + s
    r0 = s * RPS

    pltpu.sync_copy(z64_hbm.at[pl.ds(r0, RPS)], acc_m.at[pl.ds(r0, RPS)])
    pltpu.sync_copy(src_hbm.at[w], src_v)
    pltpu.sync_copy(dst_hbm.at[w], dst_v)
    plsc.subcore_barrier()

    def g_fire(j, b):
        pltpu.async_copy(xw_hbm.at[src_v.at[j]], gb.at[b], sem_g[b])

    def g_wait(b):
        pltpu.make_async_copy(xw_hbm.at[src_v.at[0]], gb.at[b], sem_g[b]).wait()

    def s_fire(j, b):
        pltpu.async_copy(gb.at[b], acc_m.at[dst_v.at[j]], sem_s[b], add=True)

    def s_wait(b):
        pltpu.make_async_copy(gb.at[b], acc_m.at[dst_v.at[0]], sem_s[b]).wait()

    def phase(j, b, drain, fire):
        b2 = (b + AHEAD) % NB
        g_wait(b)
        if drain:
            s_wait(b2)
        if fire:
            g_fire(j + AHEAD, b2)
        s_fire(j, b)

    for j in range(AHEAD):
        g_fire(j, j)
    _ring(phase, s_wait)

    plsc.subcore_barrier()
    pltpu.sync_copy(acc_m.at[pl.ds(r0, RPS)], out_m.at[c, pl.ds(r0, RPS)])


def _attr_body(dst_hbm, ea_hbm, z16f_hbm, ones_hbm, out_e, out_d,
               dst_v, ab, ones_v, acc_e, acc_d, sem_al, sem_as, sem_o):
    c = lax.axis_index("c")
    s = lax.axis_index("s")
    w = c * NS + s
    r0 = s * RPS
    ebase = w * EPW

    pltpu.sync_copy(z16f_hbm.at[pl.ds(r0, RPS)], acc_e.at[pl.ds(r0, RPS)])
    pltpu.sync_copy(z16f_hbm.at[pl.ds(r0, RPS)], acc_d.at[pl.ds(r0, RPS)])
    pltpu.sync_copy(ones_hbm, ones_v)
    pltpu.sync_copy(dst_hbm.at[w], dst_v)
    plsc.subcore_barrier()

    def al_fire(j, b):
        pltpu.async_copy(ea_hbm.at[pl.ds(ebase + j * CHUNK, CHUNK)],
                         ab.at[b], sem_al[b])

    def al_wait(b):
        pltpu.make_async_copy(ea_hbm.at[pl.ds(ebase, CHUNK)],
                              ab.at[b], sem_al[b]).wait()

    def as_fire(j, b):
        pltpu.async_copy(ab.at[b], acc_e.at[dst_v.at[j]], sem_as[b], add=True)

    def as_wait(b):
        pltpu.make_async_copy(ab.at[b], acc_e.at[dst_v.at[0]], sem_as[b]).wait()

    def o_fire(j, b):
        pltpu.async_copy(ones_v, acc_d.at[dst_v.at[j]], sem_o[b], add=True)

    def o_wait(b):
        pltpu.make_async_copy(ones_v, acc_d.at[dst_v.at[0]], sem_o[b]).wait()

    def phase(j, b, drain, fire):
        b2 = (b + AHEAD) % NB
        al_wait(b)
        if drain:
            as_wait(b2)
            o_wait(b2)
        if fire:
            al_fire(j + AHEAD, b2)
        as_fire(j, b)
        o_fire(j, b)

    def drain(b):
        as_wait(b)
        o_wait(b)

    for j in range(AHEAD):
        al_fire(j, j)
    _ring(phase, drain)

    plsc.subcore_barrier()
    pltpu.sync_copy(acc_e.at[pl.ds(r0, RPS)], out_e.at[c, pl.ds(r0, RPS)])
    pltpu.sync_copy(acc_d.at[pl.ds(r0, RPS)], out_d.at[c, pl.ds(r0, RPS)])


def _make_msg_pass():
    mesh = plsc.VectorSubcoreMesh(core_axis_name="c", subcore_axis_name="s")
    bf16 = jnp.bfloat16
    return pl.kernel(
        _msg_body,
        out_type=jax.ShapeDtypeStruct((NC, NP, H), bf16),
        mesh=mesh,
        scratch_types=(
            pltpu.VMEM((NCHUNK, CHUNK), jnp.int32),
            pltpu.VMEM((NCHUNK, CHUNK), jnp.int32),
            pltpu.VMEM_SHARED((NP, H), bf16),
            pltpu.VMEM((NB, CHUNK, H), bf16),
            [pltpu.SemaphoreType.DMA] * NB,
            [pltpu.SemaphoreType.DMA] * NB,
        ),
        compiler_params=pltpu.CompilerParams(use_tc_tiling_on_sc=False),
    )


def _make_attr_pass():
    mesh = plsc.VectorSubcoreMesh(core_axis_name="c", subcore_axis_name="s")
    f32 = jnp.float32
    return pl.kernel(
        _attr_body,
        out_type=(jax.ShapeDtypeStruct((NC, NP, D_E), f32),
                  jax.ShapeDtypeStruct((NC, NP, 16), f32)),
        mesh=mesh,
        scratch_types=(
            pltpu.VMEM((NCHUNK, CHUNK), jnp.int32),
            pltpu.VMEM((NB, CHUNK, D_E), f32),
            pltpu.VMEM((CHUNK, 16), f32),
            pltpu.VMEM_SHARED((NP, D_E), f32),
            pltpu.VMEM_SHARED((NP, 16), f32),
            [pltpu.SemaphoreType.DMA] * NB,
            [pltpu.SemaphoreType.DMA] * NB,
            [pltpu.SemaphoreType.DMA] * NB,
        ),
        compiler_params=pltpu.CompilerParams(use_tc_tiling_on_sc=False),
    )


def _tc0_body(x_ref, wx_ref, ws_ref, bs_ref, xw_out, selfx_out):
    x = x_ref[...]
    xw_out[...] = jnp.dot(
        x, wx_ref[...], preferred_element_type=jnp.float32
    ).astype(jnp.bfloat16)
    selfx_out[...] = (
        jnp.dot(x, ws_ref[...], preferred_element_type=jnp.float32) + bs_ref[...]
    )


def _tc_mid_body(selfx_ref, accm_ref, acce_ref, accd_ref,
                 we_ref, bm_ref, wh2_ref, ws2_ref, bs2_ref,
                 hw2_out, self2_out):
    am = accm_ref[0].astype(jnp.float32) + accm_ref[1].astype(jnp.float32)
    ae = acce_ref[0] + acce_ref[1]
    deg = accd_ref[0, :, 0:1] + accd_ref[1, :, 0:1]
    dc = jnp.maximum(deg, 1.0)
    ind = jnp.minimum(deg, 1.0)
    agg = (am + jnp.dot(ae, we_ref[...], preferred_element_type=jnp.float32)) / dc
    h1 = jnp.maximum(selfx_ref[...] + agg + ind * bm_ref[...], 0.0)
    hw2_out[...] = jnp.dot(
        h1, wh2_ref[...], preferred_element_type=jnp.float32
    ).astype(jnp.bfloat16)
    self2_out[...] = (
        jnp.dot(h1, ws2_ref[...], preferred_element_type=jnp.float32) + bs2_ref[...]
    )


def _tc_final_body(self2_ref, accm_ref, acce_ref, accd_ref, batch_ref,
                   we_ref, bm_ref, wfc_ref, bfc_ref,
                   out_ref, pooled_acc, cnt_acc):
    i = pl.program_id(0)

    am = accm_ref[0].astype(jnp.float32) + accm_ref[1].astype(jnp.float32)
    ae = acce_ref[0] + acce_ref[1]
    deg = accd_ref[0, :, 0:1] + accd_ref[1, :, 0:1]
    dc = jnp.maximum(deg, 1.0)
    ind = jnp.minimum(deg, 1.0)
    agg = (am + jnp.dot(ae, we_ref[...], preferred_element_type=jnp.float32)) / dc
    h2 = jnp.maximum(self2_ref[...] + agg + ind * bm_ref[...], 0.0)

    gids = lax.broadcasted_iota(jnp.int32, (ROW_BLK, G), 1)
    onehot = (batch_ref[...] == gids).astype(jnp.float32)
    pooled = lax.dot_general(onehot, h2, (((0,), (0,)), ((), ())),
                             preferred_element_type=jnp.float32)
    cnt = lax.dot_general(onehot, jnp.ones((ROW_BLK, 1), jnp.float32),
                          (((0,), (0,)), ((), ())),
                          preferred_element_type=jnp.float32)

    @pl.when(i == 0)
    def _():
        pooled_acc[...] = jnp.zeros_like(pooled_acc)
        cnt_acc[...] = jnp.zeros_like(cnt_acc)

    pooled_acc[...] += pooled
    cnt_acc[...] += cnt

    @pl.when(i == N_BLKS - 1)
    def _():
        mean = pooled_acc[...] / jnp.maximum(cnt_acc[...], 1.0)
        out_ref[...] = (
            jnp.dot(mean, wfc_ref[...], preferred_element_type=jnp.float32)
            + bfc_ref[...]
        )


def _row_spec(width):
    return pl.BlockSpec((ROW_BLK, width), lambda i: (i, 0))


def _part_spec(width):
    return pl.BlockSpec((NC, ROW_BLK, width), lambda i: (0, i, 0))


def _full_spec(shape):
    return pl.BlockSpec(shape, lambda i: tuple(0 for _ in shape))


@jax.jit
def kernel(x, edge_index, edge_attr, batch, W_msg1, b_msg1, W_self1, b_self1,
           W_msg2, b_msg2, W_self2, b_self2, W_fc, b_fc):
    f32 = jnp.float32
    src = edge_index[0].reshape(NW, NCHUNK, CHUNK)
    dst = edge_index[1].reshape(NW, NCHUNK, CHUNK)
    Wx1 = W_msg1[:D_IN]
    We1 = W_msg1[D_IN:]
    Wh2 = W_msg2[:H]
    We2 = W_msg2[H:]
    z64bf = jnp.zeros((NP, H), jnp.bfloat16)
    z16f = jnp.zeros((NP, 16), f32)
    ones_col = jnp.zeros((CHUNK, 16), f32).at[:, 0].set(1.0)
    batch2d = batch.reshape(N, 1)

    xw1, selfx = pl.pallas_call(
        _tc0_body,
        grid=(N_BLKS,),
        in_specs=[_row_spec(D_IN), _full_spec((D_IN, H)), _full_spec((D_IN, H)),
                  _full_spec((1, H))],
        out_specs=[_row_spec(H), _row_spec(H)],
        out_shape=[jax.ShapeDtypeStruct((N, H), jnp.bfloat16),
                   jax.ShapeDtypeStruct((N, H), f32)],
    )(x, Wx1, W_self1, b_self1.reshape(1, H))

    accm1 = _make_msg_pass()(xw1, src, dst, z64bf)
    acce, accd = _make_attr_pass()(dst, edge_attr, z16f, ones_col)

    hw2, self2 = pl.pallas_call(
        _tc_mid_body,
        grid=(N_BLKS,),
        in_specs=[_row_spec(H), _part_spec(H), _part_spec(D_E), _part_spec(16),
                  _full_spec((D_E, H)), _full_spec((1, H)),
                  _full_spec((H, H)), _full_spec((H, H)), _full_spec((1, H))],
        out_specs=[_row_spec(H), _row_spec(H)],
        out_shape=[jax.ShapeDtypeStruct((N, H), jnp.bfloat16),
                   jax.ShapeDtypeStruct((N, H), f32)],
    )(selfx, accm1, acce, accd, We1, b_msg1.reshape(1, H),
      Wh2, W_self2, b_self2.reshape(1, H))

    accm2 = _make_msg_pass()(hw2, src, dst, z64bf)

    out = pl.pallas_call(
        _tc_final_body,
        grid=(N_BLKS,),
        in_specs=[_row_spec(H), _part_spec(H), _part_spec(D_E), _part_spec(16),
                  pl.BlockSpec((ROW_BLK, 1), lambda i: (i, 0)),
                  _full_spec((D_E, H)), _full_spec((1, H)),
                  _full_spec((H, C)), _full_spec((1, C))],
        out_specs=pl.BlockSpec((G, C), lambda i: (0, 0)),
        out_shape=jax.ShapeDtypeStruct((G, C), f32),
        scratch_shapes=[pltpu.VMEM((G, H), f32), pltpu.VMEM((G, 1), f32)],
    )(self2, accm2, acce, accd, batch2d, We2, b_msg2.reshape(1, H),
      W_fc, b_fc.reshape(1, C))

    return out
